# initial kernel scaffold (unmeasured)
import jax
import jax.numpy as jnp
from jax import lax
from jax.experimental import pallas as pl
from jax.experimental.pallas import tpu as pltpu

N_DEV = 8
M = 4096
N_OUT = 8192
CHUNK = M // N_DEV
COL_TILE = 2048


def kernel(x, w_mat):
    k_per = x.shape[1]

    def body(x_ref, w_ref, out_ref, send_buf, recv_buf,
             send_sem, recv_sem, store_sem, credit_sem):
        my = lax.axis_index("i")
        left = lax.rem(my + N_DEV - 1, N_DEV)
        right = lax.rem(my + 1, N_DEV)

        def accum_partial(c, add_recv):
            xc = x_ref[pl.ds(c * CHUNK, CHUNK), :]
            for j in range(N_OUT // COL_TILE):
                js = slice(j * COL_TILE, (j + 1) * COL_TILE)
                p = jnp.dot(xc, w_ref[:, js],
                            preferred_element_type=jnp.float32)
                if add_recv:
                    p = p + recv_buf[:, js]
                send_buf[:, js] = p

        def make_rdma():
            return pltpu.make_async_remote_copy(
                src_ref=send_buf, dst_ref=recv_buf,
                send_sem=send_sem, recv_sem=recv_sem,
                device_id=(right,), device_id_type=pl.DeviceIdType.MESH,
            )

        def credit_to_left():
            pl.semaphore_signal(credit_sem, inc=1, device_id=(left,),
                                device_id_type=pl.DeviceIdType.MESH)

        barrier = pltpu.get_barrier_semaphore()
        for nbr in (left, right):
            pl.semaphore_signal(barrier, inc=1, device_id=(nbr,),
                                device_id_type=pl.DeviceIdType.MESH)
        pl.semaphore_wait(barrier, 2)

        accum_partial(my, add_recv=False)
        for s in range(N_DEV - 1):
            if s > 0:
                pl.semaphore_wait(credit_sem, 1)
            rdma = make_rdma()
            rdma.start()
            rdma.wait()
            c = lax.rem(my - s - 1 + 2 * N_DEV, N_DEV)
            accum_partial(c, add_recv=True)
            credit_to_left()

        own = lax.rem(my + 1, N_DEV)
        send_buf[...] = send_buf[...] * jax.nn.sigmoid(send_buf[...])
        cp = pltpu.make_async_copy(
            send_buf, out_ref.at[pl.ds(own * CHUNK, CHUNK), :], store_sem)
        cp.start()
        cp.wait()

        for t in range(N_DEV - 1):
            pl.semaphore_wait(credit_sem, 1)
            rdma = make_rdma()
            rdma.start()
            rdma.wait()
            c = lax.rem(my - t + 2 * N_DEV, N_DEV)
            cp = pltpu.make_async_copy(
                recv_buf, out_ref.at[pl.ds(c * CHUNK, CHUNK), :], store_sem)
            cp.start()
            cp.wait()
            send_buf[...] = recv_buf[...]
            credit_to_left()

        pl.semaphore_wait(credit_sem, 1)

    return pl.pallas_call(
        body,
        out_shape=jax.ShapeDtypeStruct((M, N_OUT), jnp.float32),
        in_specs=[pl.BlockSpec(memory_space=pltpu.VMEM),
                  pl.BlockSpec(memory_space=pltpu.VMEM)],
        out_specs=pl.BlockSpec(memory_space=pltpu.ANY),
        scratch_shapes=[
            pltpu.VMEM((CHUNK, N_OUT), jnp.float32),
            pltpu.VMEM((CHUNK, N_OUT), jnp.float32),
            pltpu.SemaphoreType.DMA,
            pltpu.SemaphoreType.DMA,
            pltpu.SemaphoreType.DMA,
            pltpu.SemaphoreType.REGULAR,
        ],
        compiler_params=pltpu.CompilerParams(collective_id=0),
    )(x.astype(jnp.bfloat16), w_mat.astype(jnp.bfloat16))


# baseline (device time: 2759687 ns/iter reference)
import jax
import jax.numpy as jnp
from jax import lax
from jax.experimental import pallas as pl
from jax.experimental.pallas import tpu as pltpu

N_DEV = 8
M = 4096
N_OUT = 8192
CHUNK = M // N_DEV
COL_TILE = 2048


def kernel(x, w_mat):
    k_per = x.shape[1]

    def body(x_ref, w_ref, out_ref, send_buf, recv_buf,
             send_sem, recv_sem, store_sem, credit_sem):
        my = lax.axis_index("i")
        left = lax.rem(my + N_DEV - 1, N_DEV)
        right = lax.rem(my + 1, N_DEV)

        def accum_partial(c, add_recv):
            xc = x_ref[pl.ds(c * CHUNK, CHUNK), :]
            for j in range(N_OUT // COL_TILE):
                js = slice(j * COL_TILE, (j + 1) * COL_TILE)
                p = jnp.dot(xc, w_ref[:, js],
                            preferred_element_type=jnp.float32)
                if add_recv:
                    p = p + recv_buf[:, js]
                send_buf[:, js] = p

        def make_rdma():
            return pltpu.make_async_remote_copy(
                src_ref=send_buf, dst_ref=recv_buf,
                send_sem=send_sem, recv_sem=recv_sem,
                device_id=(right,), device_id_type=pl.DeviceIdType.MESH,
            )

        def credit_to_left():
            pl.semaphore_signal(credit_sem, inc=1, device_id=(left,),
                                device_id_type=pl.DeviceIdType.MESH)

        barrier = pltpu.get_barrier_semaphore()
        for nbr in (left, right):
            pl.semaphore_signal(barrier, inc=1, device_id=(nbr,),
                                device_id_type=pl.DeviceIdType.MESH)
        pl.semaphore_wait(barrier, 2)

        accum_partial(my, add_recv=False)
        for s in range(N_DEV - 1):
            if s > 0:
                pl.semaphore_wait(credit_sem, 1)
            rdma = make_rdma()
            rdma.start()
            rdma.wait()
            c = lax.rem(my - s - 1 + 2 * N_DEV, N_DEV)
            accum_partial(c, add_recv=True)
            credit_to_left()

        own = lax.rem(my + 1, N_DEV)
        send_buf[...] = send_buf[...] * jax.nn.sigmoid(send_buf[...])
        cp = pltpu.make_async_copy(
            send_buf, out_ref.at[pl.ds(own * CHUNK, CHUNK), :], store_sem)
        cp.start()
        cp.wait()

        for t in range(N_DEV - 1):
            pl.semaphore_wait(credit_sem, 1)
            rdma = make_rdma()
            rdma.start()
            rdma.wait()
            c = lax.rem(my - t + 2 * N_DEV, N_DEV)
            cp = pltpu.make_async_copy(
                recv_buf, out_ref.at[pl.ds(c * CHUNK, CHUNK), :], store_sem)
            cp.start()
            cp.wait()
            send_buf[...] = recv_buf[...]
            credit_to_left()

        pl.semaphore_wait(credit_sem, 1)

    return pl.pallas_call(
        body,
        out_shape=jax.ShapeDtypeStruct((M, N_OUT), jnp.float32),
        in_specs=[pl.BlockSpec(memory_space=pltpu.VMEM),
                  pl.BlockSpec(memory_space=pltpu.VMEM)],
        out_specs=pl.BlockSpec(memory_space=pl.ANY),
        scratch_shapes=[
            pltpu.VMEM((CHUNK, N_OUT), jnp.float32),
            pltpu.VMEM((CHUNK, N_OUT), jnp.float32),
            pltpu.SemaphoreType.DMA,
            pltpu.SemaphoreType.DMA,
            pltpu.SemaphoreType.DMA,
            pltpu.SemaphoreType.REGULAR,
        ],
        compiler_params=pltpu.CompilerParams(collective_id=0),
    )(x.astype(jnp.bfloat16), w_mat.astype(jnp.bfloat16))


# device time: 1498645 ns/iter; 1.8415x vs baseline; 1.8415x over previous
import jax
import jax.numpy as jnp
from jax import lax
from jax.experimental import pallas as pl
from jax.experimental.pallas import tpu as pltpu

N_DEV = 8
M = 4096
N_OUT = 8192
CHUNK = M // N_DEV
HALF = N_OUT // 2
COL_TILE = 2048


def kernel(x, w_mat):
    def body(x_ref, w_ref, out_ref,
             send_cw, recv_cw, send_ccw, recv_ccw,
             send_sem_cw, recv_sem_cw, send_sem_ccw, recv_sem_ccw,
             store_sem_cw, store_sem_ccw, credit_cw, credit_ccw):
        my = lax.axis_index("i")
        left = lax.rem(my + N_DEV - 1, N_DEV)
        right = lax.rem(my + 1, N_DEV)

        def accum(c, sbuf, rbuf, col0, add_recv):
            xc = x_ref[pl.ds(c * CHUNK, CHUNK), :]
            for j in range(HALF // COL_TILE):
                js = slice(col0 + j * COL_TILE, col0 + (j + 1) * COL_TILE)
                bs = slice(j * COL_TILE, (j + 1) * COL_TILE)
                p = jnp.dot(xc, w_ref[:, js],
                            preferred_element_type=jnp.float32)
                if add_recv:
                    p = p + rbuf[:, bs]
                sbuf[:, bs] = p

        def rdma_cw():
            return pltpu.make_async_remote_copy(
                src_ref=send_cw, dst_ref=recv_cw,
                send_sem=send_sem_cw, recv_sem=recv_sem_cw,
                device_id=(right,), device_id_type=pl.DeviceIdType.MESH)

        def rdma_ccw():
            return pltpu.make_async_remote_copy(
                src_ref=send_ccw, dst_ref=recv_ccw,
                send_sem=send_sem_ccw, recv_sem=recv_sem_ccw,
                device_id=(left,), device_id_type=pl.DeviceIdType.MESH)

        def credit(sem, to):
            pl.semaphore_signal(sem, inc=1, device_id=(to,),
                                device_id_type=pl.DeviceIdType.MESH)

        barrier = pltpu.get_barrier_semaphore()
        for nbr in (left, right):
            pl.semaphore_signal(barrier, inc=1, device_id=(nbr,),
                                device_id_type=pl.DeviceIdType.MESH)
        pl.semaphore_wait(barrier, 2)

        accum(my, send_cw, recv_cw, 0, add_recv=False)
        accum(my, send_ccw, recv_ccw, HALF, add_recv=False)
        for s in range(N_DEV - 1):
            if s > 0:
                pl.semaphore_wait(credit_cw, 1)
                pl.semaphore_wait(credit_ccw, 1)
            r1 = rdma_cw()
            r2 = rdma_ccw()
            r1.start()
            r2.start()
            r1.wait()
            accum(lax.rem(my - s - 1 + 2 * N_DEV, N_DEV),
                  send_cw, recv_cw, 0, add_recv=True)
            credit(credit_cw, left)
            r2.wait()
            accum(lax.rem(my + s + 1, N_DEV),
                  send_ccw, recv_ccw, HALF, add_recv=True)
            credit(credit_ccw, right)

        own_cw = lax.rem(my + 1, N_DEV)
        own_ccw = lax.rem(my + N_DEV - 1, N_DEV)
        send_cw[...] = send_cw[...] * jax.nn.sigmoid(send_cw[...])
        send_ccw[...] = send_ccw[...] * jax.nn.sigmoid(send_ccw[...])
        cp1 = pltpu.make_async_copy(
            send_cw, out_ref.at[pl.ds(own_cw * CHUNK, CHUNK), :HALF],
            store_sem_cw)
        cp2 = pltpu.make_async_copy(
            send_ccw, out_ref.at[pl.ds(own_ccw * CHUNK, CHUNK), HALF:],
            store_sem_ccw)
        cp1.start()
        cp2.start()
        cp1.wait()
        cp2.wait()

        for t in range(N_DEV - 1):
            pl.semaphore_wait(credit_cw, 1)
            pl.semaphore_wait(credit_ccw, 1)
            r1 = rdma_cw()
            r2 = rdma_ccw()
            r1.start()
            r2.start()
            r1.wait()
            c1 = lax.rem(my - t + 2 * N_DEV, N_DEV)
            cp1 = pltpu.make_async_copy(
                recv_cw, out_ref.at[pl.ds(c1 * CHUNK, CHUNK), :HALF],
                store_sem_cw)
            cp1.start()
            cp1.wait()
            send_cw[...] = recv_cw[...]
            credit(credit_cw, left)
            r2.wait()
            c2 = lax.rem(my + t, N_DEV)
            cp2 = pltpu.make_async_copy(
                recv_ccw, out_ref.at[pl.ds(c2 * CHUNK, CHUNK), HALF:],
                store_sem_ccw)
            cp2.start()
            cp2.wait()
            send_ccw[...] = recv_ccw[...]
            credit(credit_ccw, right)

        pl.semaphore_wait(credit_cw, 1)
        pl.semaphore_wait(credit_ccw, 1)

    return pl.pallas_call(
        body,
        out_shape=jax.ShapeDtypeStruct((M, N_OUT), jnp.float32),
        in_specs=[pl.BlockSpec(memory_space=pltpu.MemorySpace.VMEM),
                  pl.BlockSpec(memory_space=pltpu.MemorySpace.VMEM)],
        out_specs=pl.BlockSpec(memory_space=pl.ANY),
        scratch_shapes=[
            pltpu.VMEM((CHUNK, HALF), jnp.float32),
            pltpu.VMEM((CHUNK, HALF), jnp.float32),
            pltpu.VMEM((CHUNK, HALF), jnp.float32),
            pltpu.VMEM((CHUNK, HALF), jnp.float32),
            pltpu.SemaphoreType.DMA,
            pltpu.SemaphoreType.DMA,
            pltpu.SemaphoreType.DMA,
            pltpu.SemaphoreType.DMA,
            pltpu.SemaphoreType.DMA,
            pltpu.SemaphoreType.DMA,
            pltpu.SemaphoreType.REGULAR,
            pltpu.SemaphoreType.REGULAR,
        ],
        compiler_params=pltpu.CompilerParams(collective_id=0),
    )(x.astype(jnp.bfloat16), w_mat.astype(jnp.bfloat16))


# device time: 858990 ns/iter; 3.2127x vs baseline; 1.7447x over previous
import jax
import jax.numpy as jnp
from jax import lax
from jax.experimental import pallas as pl
from jax.experimental.pallas import tpu as pltpu

N_DEV = 8
M = 4096
N_OUT = 8192
CHUNK = M // N_DEV
HALF = N_OUT // 2
COL_TILE = 2048


def kernel(x, w_mat):
    def body(x_ref, w_ref, out_ref,
             send_cw, recv_cw, send_ccw, recv_ccw, stage_cw, stage_ccw,
             send_sem_cw, recv_sem_cw, send_sem_ccw, recv_sem_ccw,
             store_sem_cw, store_sem_ccw, credit_cw, credit_ccw):
        my = lax.axis_index("i")
        left = lax.rem(my + N_DEV - 1, N_DEV)
        right = lax.rem(my + 1, N_DEV)

        def accum(c, sbuf, rbuf, col0, add_recv):
            xc = x_ref[pl.ds(c * CHUNK, CHUNK), :]
            for j in range(HALF // COL_TILE):
                js = slice(col0 + j * COL_TILE, col0 + (j + 1) * COL_TILE)
                bs = slice(j * COL_TILE, (j + 1) * COL_TILE)
                p = jnp.dot(xc, w_ref[:, js],
                            preferred_element_type=jnp.float32)
                if add_recv:
                    p = p + rbuf[:, bs].astype(jnp.float32)
                sbuf[:, bs] = p.astype(jnp.bfloat16)

        def rdma_cw():
            return pltpu.make_async_remote_copy(
                src_ref=send_cw, dst_ref=recv_cw,
                send_sem=send_sem_cw, recv_sem=recv_sem_cw,
                device_id=(right,), device_id_type=pl.DeviceIdType.MESH)

        def rdma_ccw():
            return pltpu.make_async_remote_copy(
                src_ref=send_ccw, dst_ref=recv_ccw,
                send_sem=send_sem_ccw, recv_sem=recv_sem_ccw,
                device_id=(left,), device_id_type=pl.DeviceIdType.MESH)

        def credit(sem, to):
            pl.semaphore_signal(sem, inc=1, device_id=(to,),
                                device_id_type=pl.DeviceIdType.MESH)

        barrier = pltpu.get_barrier_semaphore()
        for nbr in (left, right):
            pl.semaphore_signal(barrier, inc=1, device_id=(nbr,),
                                device_id_type=pl.DeviceIdType.MESH)
        pl.semaphore_wait(barrier, 2)

        accum(my, send_cw, recv_cw, 0, add_recv=False)
        accum(my, send_ccw, recv_ccw, HALF, add_recv=False)
        for s in range(N_DEV - 1):
            if s > 0:
                pl.semaphore_wait(credit_cw, 1)
                pl.semaphore_wait(credit_ccw, 1)
            r1 = rdma_cw()
            r2 = rdma_ccw()
            r1.start()
            r2.start()
            r1.wait()
            accum(lax.rem(my - s - 1 + 2 * N_DEV, N_DEV),
                  send_cw, recv_cw, 0, add_recv=True)
            credit(credit_cw, left)
            r2.wait()
            accum(lax.rem(my + s + 1, N_DEV),
                  send_ccw, recv_ccw, HALF, add_recv=True)
            credit(credit_ccw, right)

        own_cw = lax.rem(my + 1, N_DEV)
        own_ccw = lax.rem(my + N_DEV - 1, N_DEV)
        y1 = send_cw[...].astype(jnp.float32)
        y1 = y1 * jax.nn.sigmoid(y1)
        stage_cw[...] = y1
        send_cw[...] = y1.astype(jnp.bfloat16)
        y2 = send_ccw[...].astype(jnp.float32)
        y2 = y2 * jax.nn.sigmoid(y2)
        stage_ccw[...] = y2
        send_ccw[...] = y2.astype(jnp.bfloat16)
        cp1 = pltpu.make_async_copy(
            stage_cw, out_ref.at[pl.ds(own_cw * CHUNK, CHUNK), :HALF],
            store_sem_cw)
        cp2 = pltpu.make_async_copy(
            stage_ccw, out_ref.at[pl.ds(own_ccw * CHUNK, CHUNK), HALF:],
            store_sem_ccw)
        cp1.start()
        cp2.start()
        cp1.wait()
        cp2.wait()

        for t in range(N_DEV - 1):
            pl.semaphore_wait(credit_cw, 1)
            pl.semaphore_wait(credit_ccw, 1)
            r1 = rdma_cw()
            r2 = rdma_ccw()
            r1.start()
            r2.start()
            r1.wait()
            c1 = lax.rem(my - t + 2 * N_DEV, N_DEV)
            stage_cw[...] = recv_cw[...].astype(jnp.float32)
            send_cw[...] = recv_cw[...]
            credit(credit_cw, left)
            cp1 = pltpu.make_async_copy(
                stage_cw, out_ref.at[pl.ds(c1 * CHUNK, CHUNK), :HALF],
                store_sem_cw)
            cp1.start()
            r2.wait()
            c2 = lax.rem(my + t, N_DEV)
            stage_ccw[...] = recv_ccw[...].astype(jnp.float32)
            send_ccw[...] = recv_ccw[...]
            credit(credit_ccw, right)
            cp2 = pltpu.make_async_copy(
                stage_ccw, out_ref.at[pl.ds(c2 * CHUNK, CHUNK), HALF:],
                store_sem_ccw)
            cp2.start()
            cp1.wait()
            cp2.wait()

        pl.semaphore_wait(credit_cw, 1)
        pl.semaphore_wait(credit_ccw, 1)

    return pl.pallas_call(
        body,
        out_shape=jax.ShapeDtypeStruct((M, N_OUT), jnp.float32),
        in_specs=[pl.BlockSpec(memory_space=pltpu.MemorySpace.VMEM),
                  pl.BlockSpec(memory_space=pltpu.MemorySpace.VMEM)],
        out_specs=pl.BlockSpec(memory_space=pl.ANY),
        scratch_shapes=[
            pltpu.VMEM((CHUNK, HALF), jnp.bfloat16),
            pltpu.VMEM((CHUNK, HALF), jnp.bfloat16),
            pltpu.VMEM((CHUNK, HALF), jnp.bfloat16),
            pltpu.VMEM((CHUNK, HALF), jnp.bfloat16),
            pltpu.VMEM((CHUNK, HALF), jnp.float32),
            pltpu.VMEM((CHUNK, HALF), jnp.float32),
            pltpu.SemaphoreType.DMA,
            pltpu.SemaphoreType.DMA,
            pltpu.SemaphoreType.DMA,
            pltpu.SemaphoreType.DMA,
            pltpu.SemaphoreType.DMA,
            pltpu.SemaphoreType.DMA,
            pltpu.SemaphoreType.REGULAR,
            pltpu.SemaphoreType.REGULAR,
        ],
        compiler_params=pltpu.CompilerParams(collective_id=0),
    )(x.astype(jnp.bfloat16), w_mat.astype(jnp.bfloat16))


# device time: 818817 ns/iter; 3.3703x vs baseline; 1.0491x over previous
import jax
import jax.numpy as jnp
from jax import lax
from jax.experimental import pallas as pl
from jax.experimental.pallas import tpu as pltpu

N_DEV = 8
M = 4096
N_OUT = 8192
CHUNK = M // N_DEV
HALF = N_OUT // 2
COL_TILE = 2048


def kernel(x, w_mat):
    def body(x_ref, w_ref, out_ref,
             send_cw, recv_cw, send_ccw, recv_ccw, stage_cw, stage_ccw,
             send_sem_cw, recv_sem_cw, send_sem_ccw, recv_sem_ccw,
             store_sem_cw, store_sem_ccw, credit_cw, credit_ccw):
        my = lax.axis_index("i")
        left = lax.rem(my + N_DEV - 1, N_DEV)
        right = lax.rem(my + 1, N_DEV)

        def accum(c, sbuf, rbuf, col0, add_recv):
            xc = x_ref[pl.ds(c * CHUNK, CHUNK), :]
            for j in range(HALF // COL_TILE):
                js = slice(col0 + j * COL_TILE, col0 + (j + 1) * COL_TILE)
                bs = slice(j * COL_TILE, (j + 1) * COL_TILE)
                p = jnp.dot(xc, w_ref[:, js],
                            preferred_element_type=jnp.float32)
                if add_recv:
                    p = p + rbuf[:, bs].astype(jnp.float32)
                sbuf[:, bs] = p.astype(jnp.bfloat16)

        def rdma_cw():
            return pltpu.make_async_remote_copy(
                src_ref=send_cw, dst_ref=recv_cw,
                send_sem=send_sem_cw, recv_sem=recv_sem_cw,
                device_id=(right,), device_id_type=pl.DeviceIdType.MESH)

        def rdma_ccw():
            return pltpu.make_async_remote_copy(
                src_ref=send_ccw, dst_ref=recv_ccw,
                send_sem=send_sem_ccw, recv_sem=recv_sem_ccw,
                device_id=(left,), device_id_type=pl.DeviceIdType.MESH)

        def credit(sem, to):
            pl.semaphore_signal(sem, inc=1, device_id=(to,),
                                device_id_type=pl.DeviceIdType.MESH)

        barrier = pltpu.get_barrier_semaphore()
        for nbr in (left, right):
            pl.semaphore_signal(barrier, inc=1, device_id=(nbr,),
                                device_id_type=pl.DeviceIdType.MESH)
        pl.semaphore_wait(barrier, 2)

        accum(my, send_cw, recv_cw, 0, add_recv=False)
        accum(my, send_ccw, recv_ccw, HALF, add_recv=False)
        r1 = rdma_cw()
        r2 = rdma_ccw()
        r1.start()
        r2.start()
        for s in range(N_DEV - 1):
            r1.wait()
            accum(lax.rem(my - s - 1 + 2 * N_DEV, N_DEV),
                  send_cw, recv_cw, 0, add_recv=True)
            credit(credit_cw, left)
            if s < N_DEV - 2:
                pl.semaphore_wait(credit_cw, 1)
                r1 = rdma_cw()
                r1.start()
            r2.wait()
            accum(lax.rem(my + s + 1, N_DEV),
                  send_ccw, recv_ccw, HALF, add_recv=True)
            credit(credit_ccw, right)
            if s < N_DEV - 2:
                pl.semaphore_wait(credit_ccw, 1)
                r2 = rdma_ccw()
                r2.start()

        own_cw = lax.rem(my + 1, N_DEV)
        own_ccw = lax.rem(my + N_DEV - 1, N_DEV)
        y1 = send_cw[...].astype(jnp.float32)
        y1 = y1 * jax.nn.sigmoid(y1)
        stage_cw[...] = y1
        send_cw[...] = y1.astype(jnp.bfloat16)
        y2 = send_ccw[...].astype(jnp.float32)
        y2 = y2 * jax.nn.sigmoid(y2)
        stage_ccw[...] = y2
        send_ccw[...] = y2.astype(jnp.bfloat16)
        cp1 = pltpu.make_async_copy(
            stage_cw, out_ref.at[pl.ds(own_cw * CHUNK, CHUNK), :HALF],
            store_sem_cw)
        cp2 = pltpu.make_async_copy(
            stage_ccw, out_ref.at[pl.ds(own_ccw * CHUNK, CHUNK), HALF:],
            store_sem_ccw)
        cp1.start()
        cp2.start()

        pl.semaphore_wait(credit_cw, 1)
        r1 = rdma_cw()
        r1.start()
        pl.semaphore_wait(credit_ccw, 1)
        r2 = rdma_ccw()
        r2.start()
        cp1.wait()
        cp2.wait()
        for t in range(N_DEV - 1):
            r1.wait()
            c1 = lax.rem(my - t + 2 * N_DEV, N_DEV)
            stage_cw[...] = recv_cw[...].astype(jnp.float32)
            send_cw[...] = recv_cw[...]
            credit(credit_cw, left)
            cp1 = pltpu.make_async_copy(
                stage_cw, out_ref.at[pl.ds(c1 * CHUNK, CHUNK), :HALF],
                store_sem_cw)
            cp1.start()
            if t < N_DEV - 2:
                pl.semaphore_wait(credit_cw, 1)
                r1 = rdma_cw()
                r1.start()
            r2.wait()
            c2 = lax.rem(my + t, N_DEV)
            stage_ccw[...] = recv_ccw[...].astype(jnp.float32)
            send_ccw[...] = recv_ccw[...]
            credit(credit_ccw, right)
            cp2 = pltpu.make_async_copy(
                stage_ccw, out_ref.at[pl.ds(c2 * CHUNK, CHUNK), HALF:],
                store_sem_ccw)
            cp2.start()
            if t < N_DEV - 2:
                pl.semaphore_wait(credit_ccw, 1)
                r2 = rdma_ccw()
                r2.start()
            cp1.wait()
            cp2.wait()

        pl.semaphore_wait(credit_cw, 1)
        pl.semaphore_wait(credit_ccw, 1)

    return pl.pallas_call(
        body,
        out_shape=jax.ShapeDtypeStruct((M, N_OUT), jnp.float32),
        in_specs=[pl.BlockSpec(memory_space=pltpu.MemorySpace.VMEM),
                  pl.BlockSpec(memory_space=pltpu.MemorySpace.VMEM)],
        out_specs=pl.BlockSpec(memory_space=pl.ANY),
        scratch_shapes=[
            pltpu.VMEM((CHUNK, HALF), jnp.bfloat16),
            pltpu.VMEM((CHUNK, HALF), jnp.bfloat16),
            pltpu.VMEM((CHUNK, HALF), jnp.bfloat16),
            pltpu.VMEM((CHUNK, HALF), jnp.bfloat16),
            pltpu.VMEM((CHUNK, HALF), jnp.float32),
            pltpu.VMEM((CHUNK, HALF), jnp.float32),
            pltpu.SemaphoreType.DMA,
            pltpu.SemaphoreType.DMA,
            pltpu.SemaphoreType.DMA,
            pltpu.SemaphoreType.DMA,
            pltpu.SemaphoreType.DMA,
            pltpu.SemaphoreType.DMA,
            pltpu.SemaphoreType.REGULAR,
            pltpu.SemaphoreType.REGULAR,
        ],
        compiler_params=pltpu.CompilerParams(collective_id=0),
    )(x.astype(jnp.bfloat16), w_mat.astype(jnp.bfloat16))


# device time: 755657 ns/iter; 3.6520x vs baseline; 1.0836x over previous
import jax
import jax.numpy as jnp
from jax import lax
from jax.experimental import pallas as pl
from jax.experimental.pallas import tpu as pltpu

N_DEV = 8
M = 4096
N_OUT = 8192
CHUNK = M // N_DEV
SUB = N_OUT // 4
STREAMS = ((True, 0), (False, 2 * SUB), (True, SUB), (False, 3 * SUB))


def kernel(x, w_mat):
    def body(x_ref, w_ref, out_ref, send_bufs, recv_bufs, stages,
             send_sems, recv_sems, store_sems, credit_sems):
        my = lax.axis_index("i")
        left = lax.rem(my + N_DEV - 1, N_DEV)
        right = lax.rem(my + 1, N_DEV)

        def accum(c, k, col0, add_recv):
            xc = x_ref[pl.ds(c * CHUNK, CHUNK), :]
            p = jnp.dot(xc, w_ref[:, col0:col0 + SUB],
                        preferred_element_type=jnp.float32)
            if add_recv:
                p = p + recv_bufs[k].astype(jnp.float32)
            send_bufs[k, :, :] = p.astype(jnp.bfloat16)

        def rdma(k, cw):
            return pltpu.make_async_remote_copy(
                src_ref=send_bufs.at[k], dst_ref=recv_bufs.at[k],
                send_sem=send_sems.at[k], recv_sem=recv_sems.at[k],
                device_id=(right if cw else left,),
                device_id_type=pl.DeviceIdType.MESH)

        def credit(k, cw):
            pl.semaphore_signal(credit_sems.at[k], inc=1,
                                device_id=(left if cw else right,),
                                device_id_type=pl.DeviceIdType.MESH)

        barrier = pltpu.get_barrier_semaphore()
        for nbr in (left, right):
            pl.semaphore_signal(barrier, inc=1, device_id=(nbr,),
                                device_id_type=pl.DeviceIdType.MESH)
        pl.semaphore_wait(barrier, 2)

        rd = [None] * 4
        for k, (cw, col0) in enumerate(STREAMS):
            accum(my, k, col0, add_recv=False)
            rd[k] = rdma(k, cw)
            rd[k].start()
        for s in range(N_DEV - 1):
            for k, (cw, col0) in enumerate(STREAMS):
                rd[k].wait()
                c = (lax.rem(my - s - 1 + 2 * N_DEV, N_DEV) if cw
                     else lax.rem(my + s + 1, N_DEV))
                accum(c, k, col0, add_recv=True)
                credit(k, cw)
                if s < N_DEV - 2:
                    pl.semaphore_wait(credit_sems.at[k], 1)
                    rd[k] = rdma(k, cw)
                    rd[k].start()

        cps = [None] * 4
        for k, (cw, col0) in enumerate(STREAMS):
            own = lax.rem(my + 1, N_DEV) if cw else lax.rem(
                my + N_DEV - 1, N_DEV)
            y = send_bufs[k].astype(jnp.float32)
            y = y * jax.nn.sigmoid(y)
            stages[k, :, :] = y
            send_bufs[k, :, :] = y.astype(jnp.bfloat16)
            cps[k] = pltpu.make_async_copy(
                stages.at[k],
                out_ref.at[pl.ds(own * CHUNK, CHUNK), col0:col0 + SUB],
                store_sems.at[k])
            cps[k].start()

        for k, (cw, col0) in enumerate(STREAMS):
            pl.semaphore_wait(credit_sems.at[k], 1)
            rd[k] = rdma(k, cw)
            rd[k].start()
        for k in range(4):
            cps[k].wait()
        for t in range(N_DEV - 1):
            for k, (cw, col0) in enumerate(STREAMS):
                rd[k].wait()
                c = (lax.rem(my - t + 2 * N_DEV, N_DEV) if cw
                     else lax.rem(my + t, N_DEV))
                stages[k, :, :] = recv_bufs[k].astype(jnp.float32)
                send_bufs[k, :, :] = recv_bufs[k]
                credit(k, cw)
                cps[k] = pltpu.make_async_copy(
                    stages.at[k],
                    out_ref.at[pl.ds(c * CHUNK, CHUNK), col0:col0 + SUB],
                    store_sems.at[k])
                cps[k].start()
                if t < N_DEV - 2:
                    pl.semaphore_wait(credit_sems.at[k], 1)
                    rd[k] = rdma(k, cw)
                    rd[k].start()
            for k in range(4):
                cps[k].wait()

        for k in range(4):
            pl.semaphore_wait(credit_sems.at[k], 1)

    return pl.pallas_call(
        body,
        out_shape=jax.ShapeDtypeStruct((M, N_OUT), jnp.float32),
        in_specs=[pl.BlockSpec(memory_space=pltpu.MemorySpace.VMEM),
                  pl.BlockSpec(memory_space=pltpu.MemorySpace.VMEM)],
        out_specs=pl.BlockSpec(memory_space=pl.ANY),
        scratch_shapes=[
            pltpu.VMEM((4, CHUNK, SUB), jnp.bfloat16),
            pltpu.VMEM((4, CHUNK, SUB), jnp.bfloat16),
            pltpu.VMEM((4, CHUNK, SUB), jnp.float32),
            pltpu.SemaphoreType.DMA((4,)),
            pltpu.SemaphoreType.DMA((4,)),
            pltpu.SemaphoreType.DMA((4,)),
            pltpu.SemaphoreType.REGULAR((4,)),
        ],
        compiler_params=pltpu.CompilerParams(
            collective_id=0, vmem_limit_bytes=60 * 1024 * 1024),
    )(x.astype(jnp.bfloat16), w_mat.astype(jnp.bfloat16))


# device time: 745507 ns/iter; 3.7018x vs baseline; 1.0136x over previous
import jax
import jax.numpy as jnp
from jax import lax
from jax.experimental import pallas as pl
from jax.experimental.pallas import tpu as pltpu

N_DEV = 8
M = 4096
N_OUT = 8192
CHUNK = M // N_DEV
SUB = N_OUT // 4
N_HOP = 2 * (N_DEV - 1)
STREAMS = ((True, 0), (False, 2 * SUB), (True, SUB), (False, 3 * SUB))


def kernel(x, w_mat):
    def body(x_ref, w_ref, out_ref, send_bufs, recv_bufs, stages,
             send_sems, recv_sems, store_sems, credit_sems):
        my = lax.axis_index("i")
        left = lax.rem(my + N_DEV - 1, N_DEV)
        right = lax.rem(my + 1, N_DEV)

        def partial(c, col0):
            xc = x_ref[pl.ds(c * CHUNK, CHUNK), :]
            return jnp.dot(xc, w_ref[:, col0:col0 + SUB],
                           preferred_element_type=jnp.float32)

        def rdma(k, cw, slot):
            return pltpu.make_async_remote_copy(
                src_ref=send_bufs.at[k], dst_ref=recv_bufs.at[k, slot],
                send_sem=send_sems.at[k], recv_sem=recv_sems.at[k, slot],
                device_id=(right if cw else left,),
                device_id_type=pl.DeviceIdType.MESH)

        def credit(k, cw):
            pl.semaphore_signal(credit_sems.at[k], inc=1,
                                device_id=(left if cw else right,),
                                device_id_type=pl.DeviceIdType.MESH)

        def out_store(j, c, col0):
            return pltpu.make_async_copy(
                stages.at[j],
                out_ref.at[pl.ds(c * CHUNK, CHUNK), col0:col0 + SUB],
                store_sems.at[j])

        barrier = pltpu.get_barrier_semaphore()
        for nbr in (left, right):
            pl.semaphore_signal(barrier, inc=1, device_id=(nbr,),
                                device_id_type=pl.DeviceIdType.MESH)
        pl.semaphore_wait(barrier, 2)

        rd_send = [None] * 4
        cps = [None, None]
        for k, (cw, col0) in enumerate(STREAMS):
            send_bufs[k, :, :] = partial(my, col0).astype(jnp.bfloat16)
            rd_send[k] = rdma(k, cw, 0)
            rd_send[k].start()

        for h in range(N_HOP):
            slot = h % 2
            t = h - (N_DEV - 1)
            for k, (cw, col0) in enumerate(STREAMS):
                rdma(k, cw, slot).wait_recv()
                rv = recv_bufs[k, slot]
                j = k & 1
                if h < N_DEV - 1:
                    c = (lax.rem(my - h - 1 + 2 * N_DEV, N_DEV) if cw
                         else lax.rem(my + h + 1, N_DEV))
                    rd_send[k].wait_send()
                    p = partial(c, col0) + rv.astype(jnp.float32)
                    if h < N_DEV - 2:
                        send_bufs[k, :, :] = p.astype(jnp.bfloat16)
                        credit(k, cw)
                    else:
                        y = p * jax.nn.sigmoid(p)
                        if cps[j] is not None:
                            cps[j].wait()
                        stages[j, :, :] = y
                        send_bufs[k, :, :] = y.astype(jnp.bfloat16)
                        credit(k, cw)
                        cps[j] = out_store(j, c, col0)
                        cps[j].start()
                else:
                    c = (lax.rem(my - t + 2 * N_DEV, N_DEV) if cw
                         else lax.rem(my + t, N_DEV))
                    if cps[j] is not None:
                        cps[j].wait()
                    stages[j, :, :] = rv.astype(jnp.float32)
                    if h < N_HOP - 1:
                        rd_send[k].wait_send()
                        send_bufs[k, :, :] = rv
                    credit(k, cw)
                    cps[j] = out_store(j, c, col0)
                    cps[j].start()
                if h < N_HOP - 1:
                    if h + 1 >= 2:
                        pl.semaphore_wait(credit_sems.at[k], 1)
                    rd_send[k] = rdma(k, cw, (h + 1) % 2)
                    rd_send[k].start()

        for k in range(4):
            rd_send[k].wait_send()
        cps[0].wait()
        cps[1].wait()
        for k in range(4):
            pl.semaphore_wait(credit_sems.at[k], 2)

    return pl.pallas_call(
        body,
        out_shape=jax.ShapeDtypeStruct((M, N_OUT), jnp.float32),
        in_specs=[pl.BlockSpec(memory_space=pltpu.MemorySpace.VMEM),
                  pl.BlockSpec(memory_space=pltpu.MemorySpace.VMEM)],
        out_specs=pl.BlockSpec(memory_space=pl.ANY),
        scratch_shapes=[
            pltpu.VMEM((4, CHUNK, SUB), jnp.bfloat16),
            pltpu.VMEM((4, 2, CHUNK, SUB), jnp.bfloat16),
            pltpu.VMEM((2, CHUNK, SUB), jnp.float32),
            pltpu.SemaphoreType.DMA((4,)),
            pltpu.SemaphoreType.DMA((4, 2)),
            pltpu.SemaphoreType.DMA((2,)),
            pltpu.SemaphoreType.REGULAR((4,)),
        ],
        compiler_params=pltpu.CompilerParams(
            collective_id=0, vmem_limit_bytes=60 * 1024 * 1024),
    )(x.astype(jnp.bfloat16), w_mat.astype(jnp.bfloat16))


# device time: 550292 ns/iter; 5.0150x vs baseline; 1.3547x over previous
import jax
import jax.numpy as jnp
from jax import lax
from jax.experimental import pallas as pl
from jax.experimental.pallas import tpu as pltpu

N_DEV = 8
M = 4096
N_OUT = 8192
CHUNK = M // N_DEV
N_HOP = 2 * (N_DEV - 1)

CYCLES = (
    (0, 1, 2, 3, 7, 6, 5, 4),
    (0, 1, 2, 6, 5, 4, 7, 3),
    (0, 1, 5, 4, 7, 6, 2, 3),
    (0, 1, 5, 6, 2, 3, 7, 4),
    (0, 3, 2, 1, 5, 6, 7, 4),
    (0, 3, 7, 6, 2, 1, 5, 4),
)
STREAMS = []
_col = 0
for _c in (0, 2, 1, 3, 4, 5):
    _w = 768 if _c in (0, 2) else 640
    for _perm in (CYCLES[_c], tuple(reversed(CYCLES[_c]))):
        _grp = 0 if _w == 768 else 1
        _gidx = sum(1 for s in STREAMS if s[2] == _grp)
        STREAMS.append((_perm, _w, _grp, _gidx, _col))
        _col += _w
assert _col == N_OUT


def kernel(x, w_mat):
    def body(x_ref, w_ref, out_ref, send_a, recv_a, stg_a,
             send_b, recv_b, stg_b,
             send_sems, recv_sems, store_sems, credit_sems):
        my = lax.axis_index("i")
        send_g = (send_a, send_b)
        recv_g = (recv_a, recv_b)
        stg_g = (stg_a, stg_b)

        q = [None] * 12
        nxt = [None] * 12
        prv = [None] * 12
        for s, (perm, wd, g, gi, col0) in enumerate(STREAMS):
            q[s] = sum(jnp.where(my == perm[i], i, 0) for i in range(8))
            qn = lax.rem(q[s] + 1, N_DEV)
            qp = lax.rem(q[s] + N_DEV - 1, N_DEV)
            nxt[s] = sum(jnp.where(qn == i, perm[i], 0) for i in range(8))
            prv[s] = sum(jnp.where(qp == i, perm[i], 0) for i in range(8))

        def partial(p, col0, wd):
            xc = x_ref[pl.ds(p * CHUNK, CHUNK), :]
            return jnp.dot(xc, w_ref[:, col0:col0 + wd],
                           preferred_element_type=jnp.float32)

        def rdma(s):
            _, wd, g, gi, _ = STREAMS[s]
            return pltpu.make_async_remote_copy(
                src_ref=send_g[g].at[gi], dst_ref=recv_g[g].at[gi],
                send_sem=send_sems.at[s], recv_sem=recv_sems.at[s],
                device_id=(nxt[s],), device_id_type=pl.DeviceIdType.MESH)

        def out_store(s, p):
            _, wd, g, gi, col0 = STREAMS[s]
            return pltpu.make_async_copy(
                stg_g[g].at[gi],
                out_ref.at[pl.ds(p * CHUNK, CHUNK), col0:col0 + wd],
                store_sems.at[s])

        def credit(s):
            pl.semaphore_signal(credit_sems.at[s], inc=1,
                                device_id=(prv[s],),
                                device_id_type=pl.DeviceIdType.MESH)

        barrier = pltpu.get_barrier_semaphore()
        for s in range(12):
            pl.semaphore_signal(barrier, inc=1, device_id=(prv[s],),
                                device_id_type=pl.DeviceIdType.MESH)
        pl.semaphore_wait(barrier, 12)

        rd = [None] * 12
        cps = [None] * 12
        for s, (perm, wd, g, gi, col0) in enumerate(STREAMS):
            send_g[g][gi, :, :] = partial(q[s], col0, wd).astype(jnp.bfloat16)
            rd[s] = rdma(s)
            rd[s].start()

        for h in range(N_HOP):
            t = h - (N_DEV - 1)
            for s, (perm, wd, g, gi, col0) in enumerate(STREAMS):
                rd[s].wait()
                rv = recv_g[g][gi]
                if h < N_DEV - 1:
                    pr = lax.rem(q[s] - h - 1 + 2 * N_DEV, N_DEV)
                    p = partial(pr, col0, wd) + rv.astype(jnp.float32)
                    if h < N_DEV - 2:
                        send_g[g][gi, :, :] = p.astype(jnp.bfloat16)
                    else:
                        y = p * jax.nn.sigmoid(p)
                        stg_g[g][gi, :, :] = y
                        send_g[g][gi, :, :] = y.astype(jnp.bfloat16)
                        cps[s] = out_store(s, pr)
                        cps[s].start()
                else:
                    pr = lax.rem(q[s] - t + 2 * N_DEV, N_DEV)
                    cps[s].wait()
                    stg_g[g][gi, :, :] = rv.astype(jnp.float32)
                    if h < N_HOP - 1:
                        send_g[g][gi, :, :] = rv
                    cps[s] = out_store(s, pr)
                    cps[s].start()
                credit(s)
                if h < N_HOP - 1:
                    pl.semaphore_wait(credit_sems.at[s], 1)
                    rd[s] = rdma(s)
                    rd[s].start()

        for s in range(12):
            cps[s].wait()
            pl.semaphore_wait(credit_sems.at[s], 1)

    return pl.pallas_call(
        body,
        out_shape=jax.ShapeDtypeStruct((M, N_OUT), jnp.float32),
        in_specs=[pl.BlockSpec(memory_space=pltpu.MemorySpace.VMEM),
                  pl.BlockSpec(memory_space=pltpu.MemorySpace.VMEM)],
        out_specs=pl.BlockSpec(memory_space=pl.ANY),
        scratch_shapes=[
            pltpu.VMEM((4, CHUNK, 768), jnp.bfloat16),
            pltpu.VMEM((4, CHUNK, 768), jnp.bfloat16),
            pltpu.VMEM((4, CHUNK, 768), jnp.float32),
            pltpu.VMEM((8, CHUNK, 640), jnp.bfloat16),
            pltpu.VMEM((8, CHUNK, 640), jnp.bfloat16),
            pltpu.VMEM((8, CHUNK, 640), jnp.float32),
            pltpu.SemaphoreType.DMA((12,)),
            pltpu.SemaphoreType.DMA((12,)),
            pltpu.SemaphoreType.DMA((12,)),
            pltpu.SemaphoreType.REGULAR((12,)),
        ],
        compiler_params=pltpu.CompilerParams(
            collective_id=0, vmem_limit_bytes=60 * 1024 * 1024),
    )(x.astype(jnp.bfloat16), w_mat.astype(jnp.bfloat16))


# device time: 550145 ns/iter; 5.0163x vs baseline; 1.0003x over previous
import jax
import jax.numpy as jnp
from jax import lax
from jax.experimental import pallas as pl
from jax.experimental.pallas import tpu as pltpu

N_DEV = 8
M = 4096
N_OUT = 8192
CHUNK = M // N_DEV
N_HOP = 2 * (N_DEV - 1)

CYCLES = (
    (0, 1, 2, 3, 7, 6, 5, 4),
    (0, 1, 2, 6, 5, 4, 7, 3),
    (0, 1, 5, 4, 7, 6, 2, 3),
    (0, 1, 5, 6, 2, 3, 7, 4),
    (0, 3, 2, 1, 5, 6, 7, 4),
    (0, 3, 7, 6, 2, 1, 5, 4),
)
STREAMS = []
_col = 0
for _c in (0, 2, 1, 3, 4, 5):
    _w = 768 if _c in (0, 2) else 640
    for _perm in (CYCLES[_c], tuple(reversed(CYCLES[_c]))):
        _grp = 0 if _w == 768 else 1
        _gidx = sum(1 for s in STREAMS if s[2] == _grp)
        STREAMS.append((_perm, _w, _grp, _gidx, _col))
        _col += _w
assert _col == N_OUT


def kernel(x, w_mat):
    def body(x_ref, w_ref, out_ref, buf_a, stg_a, buf_b, stg_b,
             send_sems, recv_sems, store_sems, credit_sems):
        my = lax.axis_index("i")
        buf_g = (buf_a, buf_b)
        stg_g = (stg_a, stg_b)

        q = [None] * 12
        nxt = [None] * 12
        prv = [None] * 12
        for s, (perm, wd, g, gi, col0) in enumerate(STREAMS):
            q[s] = sum(jnp.where(my == perm[i], i, 0) for i in range(8))
            qn = lax.rem(q[s] + 1, N_DEV)
            qp = lax.rem(q[s] + N_DEV - 1, N_DEV)
            nxt[s] = sum(jnp.where(qn == i, perm[i], 0) for i in range(8))
            prv[s] = sum(jnp.where(qp == i, perm[i], 0) for i in range(8))

        def partial(p, col0, wd):
            xc = x_ref[pl.ds(p * CHUNK, CHUNK), :]
            return jnp.dot(xc, w_ref[:, col0:col0 + wd],
                           preferred_element_type=jnp.float32)

        def rdma(s, h):
            _, wd, g, gi, _ = STREAMS[s]
            return pltpu.make_async_remote_copy(
                src_ref=buf_g[g].at[gi, (h + 1) % 2],
                dst_ref=buf_g[g].at[gi, h % 2],
                send_sem=send_sems.at[s], recv_sem=recv_sems.at[s],
                device_id=(nxt[s],), device_id_type=pl.DeviceIdType.MESH)

        def out_store(s, p):
            _, wd, g, gi, col0 = STREAMS[s]
            return pltpu.make_async_copy(
                stg_g[g].at[gi],
                out_ref.at[pl.ds(p * CHUNK, CHUNK), col0:col0 + wd],
                store_sems.at[s])

        def credit(s):
            pl.semaphore_signal(credit_sems.at[s], inc=1,
                                device_id=(prv[s],),
                                device_id_type=pl.DeviceIdType.MESH)

        barrier = pltpu.get_barrier_semaphore()
        for s in range(12):
            pl.semaphore_signal(barrier, inc=1, device_id=(prv[s],),
                                device_id_type=pl.DeviceIdType.MESH)
        pl.semaphore_wait(barrier, 12)

        rd = [None] * 12
        cps = [None] * 12
        for s, (perm, wd, g, gi, col0) in enumerate(STREAMS):
            buf_g[g][gi, 1, :, :] = partial(q[s], col0, wd).astype(
                jnp.bfloat16)
            rd[s] = rdma(s, 0)
            rd[s].start()

        for h in range(N_HOP):
            t = h - (N_DEV - 1)
            slot = h % 2
            for s, (perm, wd, g, gi, col0) in enumerate(STREAMS):
                rd[s].wait_recv()
                rv = buf_g[g][gi, slot]
                if h < N_DEV - 1:
                    pr = lax.rem(q[s] - h - 1 + 2 * N_DEV, N_DEV)
                    p = partial(pr, col0, wd) + rv.astype(jnp.float32)
                    if h < N_DEV - 2:
                        buf_g[g][gi, slot, :, :] = p.astype(jnp.bfloat16)
                    else:
                        y = p * jax.nn.sigmoid(p)
                        stg_g[g][gi, :, :] = y
                        buf_g[g][gi, slot, :, :] = y.astype(jnp.bfloat16)
                        cps[s] = out_store(s, pr)
                        cps[s].start()
                else:
                    pr = lax.rem(q[s] - t + 2 * N_DEV, N_DEV)
                    cps[s].wait()
                    stg_g[g][gi, :, :] = rv.astype(jnp.float32)
                    cps[s] = out_store(s, pr)
                    cps[s].start()
                rd[s].wait_send()
                credit(s)
                if h < N_HOP - 1:
                    pl.semaphore_wait(credit_sems.at[s], 1)
                    rd[s] = rdma(s, h + 1)
                    rd[s].start()

        for s in range(12):
            cps[s].wait()
            pl.semaphore_wait(credit_sems.at[s], 1)

    return pl.pallas_call(
        body,
        out_shape=jax.ShapeDtypeStruct((M, N_OUT), jnp.float32),
        in_specs=[pl.BlockSpec(memory_space=pltpu.MemorySpace.VMEM),
                  pl.BlockSpec(memory_space=pltpu.MemorySpace.VMEM)],
        out_specs=pl.BlockSpec(memory_space=pl.ANY),
        scratch_shapes=[
            pltpu.VMEM((4, 2, CHUNK, 768), jnp.bfloat16),
            pltpu.VMEM((4, CHUNK, 768), jnp.float32),
            pltpu.VMEM((8, 2, CHUNK, 640), jnp.bfloat16),
            pltpu.VMEM((8, CHUNK, 640), jnp.float32),
            pltpu.SemaphoreType.DMA((12,)),
            pltpu.SemaphoreType.DMA((12,)),
            pltpu.SemaphoreType.DMA((12,)),
            pltpu.SemaphoreType.REGULAR((12,)),
        ],
        compiler_params=pltpu.CompilerParams(
            collective_id=0, vmem_limit_bytes=60 * 1024 * 1024),
    )(x.astype(jnp.bfloat16), w_mat.astype(jnp.bfloat16))
